# Initial kernel scaffold; baseline (speedup 1.0000x reference)
import jax
import jax.numpy as jnp
from jax.experimental import pallas as pl


def kernel(boxes, scores, labels):
    raise NotImplementedError("write your pallas kernel here")


# speedup vs baseline: 764.0805x; 764.0805x over previous
import jax
import jax.numpy as jnp
from jax import lax
from jax.experimental import pallas as pl
from jax.experimental.pallas import tpu as pltpu

_N = 5000
_NP = 5120
_R = 40
_NMS_SIGMA = 0.5
_NMS_SCORE = 0.001
_DETS = 300
_IOU_THRESH = 0.8


def _nms_kernel(coords_ref, s0_ref, lab_ref, out_ref,
                s_ref, supp_ref, area_ref, off_ref,
                xo1_ref, yo1_ref, xo2_ref, yo2_ref, iota_ref):
    x1 = coords_ref[0]
    y1 = coords_ref[1]
    x2 = coords_ref[2]
    y2 = coords_ref[3]

    max_coord = jnp.max(coords_ref[...]) + 1.0
    off = lab_ref[...] * max_coord
    off_ref[...] = off
    xo1_ref[...] = x1 + off
    yo1_ref[...] = y1 + off
    xo2_ref[...] = x2 + off
    yo2_ref[...] = y2 + off
    area_ref[...] = (x2 - x1) * (y2 - y1)

    s_ref[...] = s0_ref[...]
    supp_ref[...] = jnp.zeros((_R, 128), jnp.float32)
    out_ref[...] = jnp.zeros((_R, 128), jnp.float32)
    iota_ref[...] = (lax.broadcasted_iota(jnp.int32, (_R, 128), 0) * 128
                     + lax.broadcasted_iota(jnp.int32, (_R, 128), 1))

    def cond(c):
        i, kept = c
        return jnp.logical_and(i < _N, kept < _DETS)

    def body(c):
        i, kept = c
        s = s_ref[...]
        m = jnp.max(s)
        cont = m > _NMS_SCORE

        iota = iota_ref[...]
        idx = jnp.min(jnp.where(s == m, iota, _NP))
        mask = iota == idx

        def ext(p):
            return jnp.sum(jnp.where(mask, p, 0.0))

        bx1 = ext(x1)
        by1 = ext(y1)
        bx2 = ext(x2)
        by2 = ext(y2)
        boff = ext(off_ref[...])
        supv = ext(supp_ref[...])
        take = jnp.logical_and(cont, supv < 0.5)
        a1 = (bx2 - bx1) * (by2 - by1)

        @pl.when(cont)
        def _():
            ix1 = jnp.maximum(bx1 + boff, xo1_ref[...])
            iy1 = jnp.maximum(by1 + boff, yo1_ref[...])
            ix2 = jnp.minimum(bx2 + boff, xo2_ref[...])
            iy2 = jnp.minimum(by2 + boff, yo2_ref[...])
            inter = jnp.maximum(ix2 - ix1, 0.0) * jnp.maximum(iy2 - iy1, 0.0)
            iou = inter / (a1 + area_ref[...] - inter + 1e-9)
            decay = jnp.exp(-(iou * iou) / _NMS_SIGMA)
            s_ref[...] = jnp.where(mask, -jnp.inf, s * decay)

            val = jnp.where(take, m, 0.0)
            out_ref[...] = jnp.where(iota == i, val, out_ref[...])

            @pl.when(take)
            def _():
                jx1 = jnp.maximum(bx1, x1)
                jy1 = jnp.maximum(by1, y1)
                jx2 = jnp.minimum(bx2, x2)
                jy2 = jnp.minimum(by2, y2)
                jint = (jnp.maximum(jx2 - jx1, 0.0)
                        * jnp.maximum(jy2 - jy1, 0.0))
                jiou = jint / (a1 + area_ref[...] - jint + 1e-9)
                supp_ref[...] = jnp.maximum(
                    supp_ref[...], (jiou > _IOU_THRESH).astype(jnp.float32))

        i2 = jnp.where(cont, i + 1, _N)
        return (i2, kept + take.astype(jnp.int32))

    lax.while_loop(cond, body, (jnp.int32(0), jnp.int32(0)))


def kernel(boxes, scores, labels):
    pad = _NP - _N
    coords = jnp.pad(boxes, ((0, pad), (0, 0))).T.reshape(4, _R, 128)
    sp = jnp.pad(scores, (0, pad), constant_values=-jnp.inf).reshape(_R, 128)
    lp = jnp.pad(labels.astype(jnp.float32), (0, pad)).reshape(_R, 128)

    out = pl.pallas_call(
        _nms_kernel,
        out_shape=jax.ShapeDtypeStruct((_R, 128), jnp.float32),
        scratch_shapes=[pltpu.VMEM((_R, 128), jnp.float32)] * 8
                       + [pltpu.VMEM((_R, 128), jnp.int32)],
    )(coords, sp, lp)
    return out.reshape(_NP)[:_N]


# speedup vs baseline: 769.5659x; 1.0072x over previous
import jax
import jax.numpy as jnp
from jax import lax
from jax.experimental import pallas as pl
from jax.experimental.pallas import tpu as pltpu

_N = 5000
_NP = 5120
_R = 40
_NMS_SIGMA = 0.5
_NMS_SCORE = 0.001
_DETS = 300
_IOU_THRESH = 0.8


def _nms_kernel(coords_ref, packed_ref, s0_ref, lab_ref, out_ref,
                area_ref, xo1_ref, yo1_ref, xo2_ref, yo2_ref, iota_ref):
    x1 = coords_ref[0]
    y1 = coords_ref[1]
    x2 = coords_ref[2]
    y2 = coords_ref[3]

    max_coord = jnp.max(coords_ref[...]) + 1.0
    off = lab_ref[...] * max_coord
    xo1_ref[...] = x1 + off
    yo1_ref[...] = y1 + off
    xo2_ref[...] = x2 + off
    yo2_ref[...] = y2 + off
    area_ref[...] = (x2 - x1) * (y2 - y1)
    iota_ref[...] = (lax.broadcasted_iota(jnp.int32, (_R, 128), 0) * 128
                     + lax.broadcasted_iota(jnp.int32, (_R, 128), 1))

    def cond(c):
        i, kept, _, _, _ = c
        return jnp.logical_and(i < _N, kept < _DETS)

    def body(c):
        i, kept, s, supp, out = c
        iota = iota_ref[...]
        area = area_ref[...]

        m = jnp.max(s)
        cont = m > _NMS_SCORE
        idx = jnp.min(jnp.where(s == m, iota, _NP))
        mask = iota == idx

        row = packed_ref[pl.ds(idx, 1), :]
        bx1 = row[0, 0]
        by1 = row[0, 1]
        bx2 = row[0, 2]
        by2 = row[0, 3]
        boff = row[0, 4] * max_coord
        a1 = (bx2 - bx1) * (by2 - by1)

        supv = jnp.max(jnp.where(mask, supp, 0.0))
        take = jnp.logical_and(cont, supv < 0.5)

        ix1 = jnp.maximum(bx1 + boff, xo1_ref[...])
        iy1 = jnp.maximum(by1 + boff, yo1_ref[...])
        ix2 = jnp.minimum(bx2 + boff, xo2_ref[...])
        iy2 = jnp.minimum(by2 + boff, yo2_ref[...])
        inter = jnp.maximum(ix2 - ix1, 0.0) * jnp.maximum(iy2 - iy1, 0.0)
        iou = inter / (a1 + area - inter + 1e-9)
        decay = jnp.exp(-(iou * iou) / _NMS_SIGMA)
        s2 = jnp.where(mask, -1.0, s * decay)

        out2 = jnp.where(iota == i, jnp.where(take, m, 0.0), out)

        jx1 = jnp.maximum(bx1, x1)
        jy1 = jnp.maximum(by1, y1)
        jx2 = jnp.minimum(bx2, x2)
        jy2 = jnp.minimum(by2, y2)
        jint = jnp.maximum(jx2 - jx1, 0.0) * jnp.maximum(jy2 - jy1, 0.0)
        hit = (1.0 + _IOU_THRESH) * jint > _IOU_THRESH * (area + (a1 + 1e-9))
        sadd = jnp.where(jnp.logical_and(hit, take), 1.0, 0.0)
        supp2 = jnp.maximum(supp, sadd)

        i2 = jnp.where(cont, i + 1, _N)
        return (i2, kept + take.astype(jnp.int32), s2, supp2, out2)

    zeros = jnp.zeros((_R, 128), jnp.float32)
    final = lax.while_loop(
        cond, body, (jnp.int32(0), jnp.int32(0), s0_ref[...], zeros, zeros))
    out_ref[...] = final[4]


def kernel(boxes, scores, labels):
    pad = _NP - _N
    labf = labels.astype(jnp.float32)
    coords = jnp.pad(boxes, ((0, pad), (0, 0))).T.reshape(4, _R, 128)
    packed = jnp.pad(
        jnp.concatenate([boxes, labf[:, None]], axis=1),
        ((0, pad), (0, 3)))
    sp = jnp.pad(scores, (0, pad), constant_values=-1.0).reshape(_R, 128)
    lp = jnp.pad(labf, (0, pad)).reshape(_R, 128)

    out = pl.pallas_call(
        _nms_kernel,
        out_shape=jax.ShapeDtypeStruct((_R, 128), jnp.float32),
        scratch_shapes=[pltpu.VMEM((_R, 128), jnp.float32)] * 5
                       + [pltpu.VMEM((_R, 128), jnp.int32)],
    )(coords, packed, sp, lp)
    return out.reshape(_NP)[:_N]


# speedup vs baseline: 1264.0372x; 1.6425x over previous
import jax
import jax.numpy as jnp
from jax import lax
from jax.experimental import pallas as pl
from jax.experimental.pallas import tpu as pltpu

_N = 5000
_NP = 5120
_R = 40
_NMS_SIGMA = 0.5
_NMS_SCORE = 0.001
_DETS = 300
_IOU_THRESH = 0.8


def _nms_kernel(coords_ref, rows_ref, s0_ref, lab_ref, out_ref,
                xo1_ref, yo1_ref, xo2_ref, yo2_ref, area_ref, kodd_ref,
                iota_ref):
    x1 = coords_ref[0]
    y1 = coords_ref[1]
    x2 = coords_ref[2]
    y2 = coords_ref[3]

    max_coord = jnp.max(coords_ref[...]) + 1.0
    off = lab_ref[...] * max_coord
    xo1_ref[...] = x1 + off
    yo1_ref[...] = y1 + off
    xo2_ref[...] = x2 + off
    yo2_ref[...] = y2 + off
    area_ref[...] = (x2 - x1) * (y2 - y1)

    iota = (lax.broadcasted_iota(jnp.int32, (_R, 128), 0) * 128
            + lax.broadcasted_iota(jnp.int32, (_R, 128), 1))
    iota_ref[...] = iota
    k0 = iota.astype(jnp.float32) * 2.0
    kodd_ref[...] = k0 + 1.0

    def cond(c):
        i, kept, _, _, _ = c
        return jnp.logical_and(i < _N, kept < _DETS)

    def body(c):
        i, kept, s, kkey, out = c

        m = jnp.max(s)
        cont = m > _NMS_SCORE
        minkey = jnp.min(jnp.where(s == m, kkey, 3.0e7))
        ki = minkey.astype(jnp.int32)
        idx = ki >> 1
        take = jnp.logical_and(cont, (ki & 1) == 0)

        bx1 = rows_ref[0, idx]
        by1 = rows_ref[1, idx]
        bx2 = rows_ref[2, idx]
        by2 = rows_ref[3, idx]
        boff = rows_ref[4, idx] * max_coord
        a1 = (bx2 - bx1) * (by2 - by1)
        area = area_ref[...]

        ix1 = jnp.maximum(bx1 + boff, xo1_ref[...])
        iy1 = jnp.maximum(by1 + boff, yo1_ref[...])
        ix2 = jnp.minimum(bx2 + boff, xo2_ref[...])
        iy2 = jnp.minimum(by2 + boff, yo2_ref[...])
        inter = jnp.maximum(ix2 - ix1, 0.0) * jnp.maximum(iy2 - iy1, 0.0)
        iou = inter / (a1 + area - inter + 1e-9)
        decay = jnp.exp(-(iou * iou) / _NMS_SIGMA)
        s2 = jnp.where(kkey == minkey, 0.0, s * decay)

        out2 = jnp.where(iota_ref[...] == i, jnp.where(take, m, 0.0), out)

        jx1 = jnp.maximum(bx1, x1)
        jy1 = jnp.maximum(by1, y1)
        jx2 = jnp.minimum(bx2, x2)
        jy2 = jnp.minimum(by2, y2)
        jint = jnp.maximum(jx2 - jx1, 0.0) * jnp.maximum(jy2 - jy1, 0.0)
        hit = (1.0 + _IOU_THRESH) * jint > _IOU_THRESH * (area + (a1 + 1e-9))
        kkey2 = jnp.where(jnp.logical_and(hit, take), kodd_ref[...], kkey)

        i2 = jnp.where(cont, i + 1, _N)
        return (i2, kept + take.astype(jnp.int32), s2, kkey2, out2)

    final = lax.while_loop(
        cond, body, (jnp.int32(0), jnp.int32(0), s0_ref[...], k0,
                     jnp.zeros((_R, 128), jnp.float32)))
    out_ref[...] = final[4]


def kernel(boxes, scores, labels):
    pad = _NP - _N
    labf = labels.astype(jnp.float32)
    coords = jnp.pad(boxes, ((0, pad), (0, 0))).T.reshape(4, _R, 128)
    rows = jnp.pad(
        jnp.concatenate([boxes.T, labf[None, :]], axis=0), ((0, 0), (0, pad)))
    sp = jnp.pad(scores, (0, pad), constant_values=-1.0).reshape(_R, 128)
    lp = jnp.pad(labf, (0, pad)).reshape(_R, 128)

    out = pl.pallas_call(
        _nms_kernel,
        out_shape=jax.ShapeDtypeStruct((_R, 128), jnp.float32),
        in_specs=[
            pl.BlockSpec(memory_space=pltpu.VMEM),
            pl.BlockSpec(memory_space=pltpu.SMEM),
            pl.BlockSpec(memory_space=pltpu.VMEM),
            pl.BlockSpec(memory_space=pltpu.VMEM),
        ],
        scratch_shapes=[pltpu.VMEM((_R, 128), jnp.float32)] * 6
                       + [pltpu.VMEM((_R, 128), jnp.int32)],
    )(coords, rows, sp, lp)
    return out.reshape(_NP)[:_N]


# speedup vs baseline: 1398.1428x; 1.1061x over previous
import jax
import jax.numpy as jnp
from jax import lax
from jax.experimental import pallas as pl
from jax.experimental.pallas import tpu as pltpu

_N = 5000
_NP = 5120
_R = 40
_NMS_SIGMA = 0.5
_NMS_SCORE = 0.001
_DETS = 300
_IOU_THRESH = 0.8


def _nms_kernel(coords_ref, rows_ref, s0_ref, lab_ref, out_ref,
                xo1_ref, yo1_ref, xo2_ref, yo2_ref, area_ref, kodd_ref,
                iota_ref):
    x1 = coords_ref[0]
    y1 = coords_ref[1]
    x2 = coords_ref[2]
    y2 = coords_ref[3]

    max_coord = jnp.max(coords_ref[...]) + 1.0
    off = lab_ref[...] * max_coord
    xo1_ref[...] = x1 + off
    yo1_ref[...] = y1 + off
    xo2_ref[...] = x2 + off
    yo2_ref[...] = y2 + off
    area_ref[...] = (x2 - x1) * (y2 - y1)

    iota = (lax.broadcasted_iota(jnp.int32, (_R, 128), 0) * 128
            + lax.broadcasted_iota(jnp.int32, (_R, 128), 1))
    iota_ref[...] = iota
    k0 = iota.astype(jnp.float32) * 2.0
    kodd_ref[...] = k0 + 1.0

    def cond(c):
        i, kept, _, _, _ = c
        return jnp.logical_and(i < _N, kept < _DETS)

    def body(c):
        i, kept, s, kkey, out = c

        a = jnp.max(s.reshape(5, 8, 128), axis=0)
        for sh in (4, 2, 1):
            a = jnp.maximum(a, pltpu.roll(a, sh, 0))
        m8 = jnp.max(a, axis=1, keepdims=True)
        m = jnp.broadcast_to(m8[None], (5, 8, 128)).reshape(_R, 128)
        masked = jnp.where(
            jnp.logical_and(s == m, s > _NMS_SCORE), kkey, 3.0e7)
        b = jnp.min(masked.reshape(5, 8, 128), axis=0)
        for sh in (4, 2, 1):
            b = jnp.minimum(b, pltpu.roll(b, sh, 0))
        mk8 = jnp.min(b, axis=1, keepdims=True)
        minkey = mk8[0, 0]
        ki = minkey.astype(jnp.int32)
        cont = ki < 20000000
        idx = jnp.minimum(ki >> 1, _N - 1)
        take = jnp.logical_and(cont, (ki & 1) == 0)

        bx1 = rows_ref[0, idx]
        by1 = rows_ref[1, idx]
        bx2 = rows_ref[2, idx]
        by2 = rows_ref[3, idx]
        boff = rows_ref[4, idx] * max_coord
        a1 = (bx2 - bx1) * (by2 - by1)
        area = area_ref[...]

        ix1 = jnp.maximum(bx1 + boff, xo1_ref[...])
        iy1 = jnp.maximum(by1 + boff, yo1_ref[...])
        ix2 = jnp.minimum(bx2 + boff, xo2_ref[...])
        iy2 = jnp.minimum(by2 + boff, yo2_ref[...])
        inter = jnp.maximum(ix2 - ix1, 0.0) * jnp.maximum(iy2 - iy1, 0.0)
        iou = inter / (a1 + area - inter + 1e-9)
        decay = jnp.exp(-(iou * iou) / _NMS_SIGMA)
        s2 = jnp.where(kkey == minkey, 0.0, s * decay)

        out2 = jnp.where(
            jnp.logical_and(iota_ref[...] == i, take), m, out)

        jx1 = jnp.maximum(bx1, x1)
        jy1 = jnp.maximum(by1, y1)
        jx2 = jnp.minimum(bx2, x2)
        jy2 = jnp.minimum(by2, y2)
        jint = jnp.maximum(jx2 - jx1, 0.0) * jnp.maximum(jy2 - jy1, 0.0)
        hit = (1.0 + _IOU_THRESH) * jint > _IOU_THRESH * (area + (a1 + 1e-9))
        kkey2 = jnp.where(jnp.logical_and(hit, take), kodd_ref[...], kkey)

        i2 = jnp.where(cont, i + 1, _N)
        return (i2, kept + take.astype(jnp.int32), s2, kkey2, out2)

    final = lax.while_loop(
        cond, body, (jnp.int32(0), jnp.int32(0), s0_ref[...], k0,
                     jnp.zeros((_R, 128), jnp.float32)))
    out_ref[...] = final[4]


def kernel(boxes, scores, labels):
    pad = _NP - _N
    labf = labels.astype(jnp.float32)
    coords = jnp.pad(boxes, ((0, pad), (0, 0))).T.reshape(4, _R, 128)
    rows = jnp.pad(
        jnp.concatenate([boxes.T, labf[None, :]], axis=0), ((0, 0), (0, pad)))
    sp = jnp.pad(scores, (0, pad), constant_values=-1.0).reshape(_R, 128)
    lp = jnp.pad(labf, (0, pad)).reshape(_R, 128)

    out = pl.pallas_call(
        _nms_kernel,
        out_shape=jax.ShapeDtypeStruct((_R, 128), jnp.float32),
        in_specs=[
            pl.BlockSpec(memory_space=pltpu.VMEM),
            pl.BlockSpec(memory_space=pltpu.SMEM),
            pl.BlockSpec(memory_space=pltpu.VMEM),
            pl.BlockSpec(memory_space=pltpu.VMEM),
        ],
        scratch_shapes=[pltpu.VMEM((_R, 128), jnp.float32)] * 6
                       + [pltpu.VMEM((_R, 128), jnp.int32)],
    )(coords, rows, sp, lp)
    return out.reshape(_NP)[:_N]


# speedup vs baseline: 1946.5063x; 1.3922x over previous
import jax
import jax.numpy as jnp
from jax import lax
from jax.experimental import pallas as pl
from jax.experimental.pallas import tpu as pltpu

_N = 5000
_NP = 5120
_R = 40
_NMS_SIGMA = 0.5
_NMS_SCORE = 0.001
_DETS = 300
_IOU_THRESH = 0.8
_BIG = 1.0e6
_BIGI = 100000


def _fullmax(p):
    a = jnp.max(p.reshape(5, 8, 128), axis=0)
    for sh in (4, 2, 1):
        a = jnp.maximum(a, pltpu.roll(a, sh, 0))
    return jnp.max(a, axis=1, keepdims=True)


def _fullmin(p):
    a = jnp.min(p.reshape(5, 8, 128), axis=0)
    for sh in (4, 2, 1):
        a = jnp.minimum(a, pltpu.roll(a, sh, 0))
    return jnp.min(a, axis=1, keepdims=True)


def _bcast(x81):
    return jnp.broadcast_to(x81[None], (5, 8, 128)).reshape(_R, 128)


def _nms_kernel(coords_ref, rows_ref, s0_ref, lab_ref, out_ref,
                xo1_ref, yo1_ref, xo2_ref, yo2_ref, area_ref, kodd_ref,
                iota_ref):
    x1 = coords_ref[0]
    y1 = coords_ref[1]
    x2 = coords_ref[2]
    y2 = coords_ref[3]

    max_coord = jnp.max(coords_ref[...]) + 1.0
    off = lab_ref[...] * max_coord
    xo1_ref[...] = x1 + off
    yo1_ref[...] = y1 + off
    xo2_ref[...] = x2 + off
    yo2_ref[...] = y2 + off
    area_ref[...] = (x2 - x1) * (y2 - y1)

    iota = (lax.broadcasted_iota(jnp.int32, (_R, 128), 0) * 128
            + lax.broadcasted_iota(jnp.int32, (_R, 128), 1))
    iota_ref[...] = iota
    k0 = iota.astype(jnp.float32) * 2.0
    kodd_ref[...] = k0 + 1.0

    def cond(c):
        i, kept, _, _, _ = c
        return jnp.logical_and(i < _N, kept < _DETS)

    def body(c):
        i, kept, s, kkey, out = c
        area = area_ref[...]
        iotap = iota_ref[...]

        sgood = s > _NMS_SCORE
        m8 = _fullmax(s)
        m = _bcast(m8)
        maskA = jnp.logical_and(s == m, sgood)
        mk18 = _fullmin(jnp.where(maskA, kkey, _BIG))
        tk18 = _fullmax(jnp.where(maskA, kkey, -1.0))
        m28 = _fullmax(jnp.where(s == m, -1.0, s))
        m2 = _bcast(m28)
        maskB = jnp.logical_and(s == m2, sgood)
        mk28 = _fullmin(jnp.where(maskB, kkey, _BIG))
        z8 = mk28 * 2.0 + jnp.where(tk18 == mk18, 0.0, 1.0)

        ki1 = mk18[0, 0].astype(jnp.int32)
        cont1 = ki1 < _BIGI
        idx1 = jnp.minimum(ki1 >> 1, _N - 1)
        take1 = jnp.logical_and(cont1, (ki1 & 1) == 0)

        zi = z8[0, 0].astype(jnp.int32)
        tie1 = (zi & 1) == 1
        ki2 = zi >> 1
        cont2 = ki2 < _BIGI
        idx2 = jnp.minimum(ki2 >> 1, _N - 1)

        bx1 = rows_ref[0, idx1]
        by1 = rows_ref[1, idx1]
        bx2 = rows_ref[2, idx1]
        by2 = rows_ref[3, idx1]
        boff = rows_ref[4, idx1] * max_coord
        a1 = (bx2 - bx1) * (by2 - by1)

        cx1 = rows_ref[0, idx2]
        cy1 = rows_ref[1, idx2]
        cx2 = rows_ref[2, idx2]
        cy2 = rows_ref[3, idx2]
        coff = rows_ref[4, idx2] * max_coord
        a2 = (cx2 - cx1) * (cy2 - cy1)

        w12 = (jnp.minimum(bx2 + boff, cx2 + coff)
               - jnp.maximum(bx1 + boff, cx1 + coff))
        h12 = (jnp.minimum(by2 + boff, cy2 + coff)
               - jnp.maximum(by1 + boff, cy1 + coff))
        inter12 = jnp.maximum(w12, 0.0) * jnp.maximum(h12, 0.0)
        valid2 = jnp.logical_and(
            jnp.logical_and(cont1, jnp.logical_not(tie1)), inter12 == 0.0)

        jw = jnp.minimum(bx2, cx2) - jnp.maximum(bx1, cx1)
        jh = jnp.minimum(by2, cy2) - jnp.maximum(by1, cy1)
        jint12 = jnp.maximum(jw, 0.0) * jnp.maximum(jh, 0.0)
        hit12 = jnp.logical_and(
            take1,
            (1.0 + _IOU_THRESH) * jint12 > _IOU_THRESH * (a2 + (a1 + 1e-9)))
        take2 = jnp.logical_and(
            jnp.logical_and(valid2, cont2),
            jnp.logical_and((ki2 & 1) == 0, jnp.logical_not(hit12)))
        take2c = jnp.logical_and(take2, kept + take1.astype(jnp.int32) < _DETS)

        def soft_decay(px1, py1, px2, py2, poff, pa):
            ix1 = jnp.maximum(px1 + poff, xo1_ref[...])
            iy1 = jnp.maximum(py1 + poff, yo1_ref[...])
            ix2 = jnp.minimum(px2 + poff, xo2_ref[...])
            iy2 = jnp.minimum(py2 + poff, yo2_ref[...])
            inter = jnp.maximum(ix2 - ix1, 0.0) * jnp.maximum(iy2 - iy1, 0.0)
            iou = inter / (pa + area - inter + 1e-9)
            return jnp.exp(-(iou * iou) / _NMS_SIGMA)

        decay1 = soft_decay(bx1, by1, bx2, by2, boff, a1)
        decay2 = soft_decay(cx1, cy1, cx2, cy2, coff, a2)
        dd = decay1 * jnp.where(valid2, decay2, 1.0)
        zmask = jnp.logical_or(
            kkey == _bcast(mk18),
            jnp.logical_and(kkey == _bcast(mk28), valid2))
        s2 = jnp.where(zmask, 0.0, s * dd)

        out2 = jnp.where(jnp.logical_and(iotap == i, take1), m, out)
        out2 = jnp.where(jnp.logical_and(iotap == i + 1, take2c), m2, out2)

        def greedy_hit(px1, py1, px2, py2, pa):
            gx1 = jnp.maximum(px1, x1)
            gy1 = jnp.maximum(py1, y1)
            gx2 = jnp.minimum(px2, x2)
            gy2 = jnp.minimum(py2, y2)
            gint = (jnp.maximum(gx2 - gx1, 0.0)
                    * jnp.maximum(gy2 - gy1, 0.0))
            return ((1.0 + _IOU_THRESH) * gint
                    > _IOU_THRESH * (area + (pa + 1e-9)))

        sup = jnp.logical_or(
            jnp.logical_and(greedy_hit(bx1, by1, bx2, by2, a1), take1),
            jnp.logical_and(greedy_hit(cx1, cy1, cx2, cy2, a2), take2))
        kkey2 = jnp.where(sup, kodd_ref[...], kkey)

        i2 = jnp.where(
            cont1,
            jnp.where(valid2, jnp.where(cont2, i + 2, _N), i + 1),
            _N)
        kept2 = kept + take1.astype(jnp.int32) + take2c.astype(jnp.int32)
        return (i2, kept2, s2, kkey2, out2)

    final = lax.while_loop(
        cond, body, (jnp.int32(0), jnp.int32(0), s0_ref[...], k0,
                     jnp.zeros((_R, 128), jnp.float32)))
    out_ref[...] = final[4]


def kernel(boxes, scores, labels):
    pad = _NP - _N
    labf = labels.astype(jnp.float32)
    coords = jnp.pad(boxes, ((0, pad), (0, 0))).T.reshape(4, _R, 128)
    rows = jnp.pad(
        jnp.concatenate([boxes.T, labf[None, :]], axis=0), ((0, 0), (0, pad)))
    sp = jnp.pad(scores, (0, pad), constant_values=-1.0).reshape(_R, 128)
    lp = jnp.pad(labf, (0, pad)).reshape(_R, 128)

    out = pl.pallas_call(
        _nms_kernel,
        out_shape=jax.ShapeDtypeStruct((_R, 128), jnp.float32),
        in_specs=[
            pl.BlockSpec(memory_space=pltpu.VMEM),
            pl.BlockSpec(memory_space=pltpu.SMEM),
            pl.BlockSpec(memory_space=pltpu.VMEM),
            pl.BlockSpec(memory_space=pltpu.VMEM),
        ],
        scratch_shapes=[pltpu.VMEM((_R, 128), jnp.float32)] * 6
                       + [pltpu.VMEM((_R, 128), jnp.int32)],
    )(coords, rows, sp, lp)
    return out.reshape(_NP)[:_N]


# speedup vs baseline: 2328.0413x; 1.1960x over previous
import jax
import jax.numpy as jnp
from jax import lax
from jax.experimental import pallas as pl
from jax.experimental.pallas import tpu as pltpu

_N = 5000
_NP = 5120
_R = 40
_NMS_SIGMA = 0.5
_NMS_SCORE = 0.001
_DETS = 300
_IOU_THRESH = 0.8
_K = 4
_BIG = 1.0e6
_BIGI = 100000


def _fullmax(p):
    a = jnp.max(p.reshape(5, 8, 128), axis=0)
    for sh in (4, 2, 1):
        a = jnp.maximum(a, pltpu.roll(a, sh, 0))
    return jnp.max(a, axis=1, keepdims=True)


def _fullmin(p):
    a = jnp.min(p.reshape(5, 8, 128), axis=0)
    for sh in (4, 2, 1):
        a = jnp.minimum(a, pltpu.roll(a, sh, 0))
    return jnp.min(a, axis=1, keepdims=True)


def _bcast(x81):
    return jnp.broadcast_to(x81[None], (5, 8, 128)).reshape(_R, 128)


def _nms_kernel(coords_ref, rows_ref, s0_ref, lab_ref, out_ref,
                xo1_ref, yo1_ref, xo2_ref, yo2_ref, area_ref, kodd_ref,
                iota_ref):
    x1 = coords_ref[0]
    y1 = coords_ref[1]
    x2 = coords_ref[2]
    y2 = coords_ref[3]

    max_coord = jnp.max(coords_ref[...]) + 1.0
    off = lab_ref[...] * max_coord
    xo1_ref[...] = x1 + off
    yo1_ref[...] = y1 + off
    xo2_ref[...] = x2 + off
    yo2_ref[...] = y2 + off
    area_ref[...] = (x2 - x1) * (y2 - y1)

    iota = (lax.broadcasted_iota(jnp.int32, (_R, 128), 0) * 128
            + lax.broadcasted_iota(jnp.int32, (_R, 128), 1))
    iota_ref[...] = iota
    k0 = iota.astype(jnp.float32) * 2.0
    kodd_ref[...] = k0 + 1.0

    def cond(c):
        i, kept, _, _, _ = c
        return jnp.logical_and(i < _N, kept < _DETS)

    def body(c):
        i, kept, s, kkey, out = c
        area = area_ref[...]
        iotap = iota_ref[...]
        sgood = s > _NMS_SCORE

        mv8s, mk8s, z8s = [], [], []
        excl = None
        for l in range(_K):
            if l == 0:
                mv8 = _fullmax(s)
                excl = s == _bcast(mv8)
            else:
                mv8 = _fullmax(jnp.where(excl, -1.0, s))
                excl = jnp.logical_or(excl, s == _bcast(mv8))
            lvl = jnp.logical_and(s == _bcast(mv8), sgood)
            mk8 = _fullmin(jnp.where(lvl, kkey, _BIG))
            tk8 = _fullmax(jnp.where(lvl, kkey, -1.0))
            z8 = mk8 * 2.0 + jnp.where(tk8 == mk8, 0.0, 1.0)
            mv8s.append(mv8)
            mk8s.append(mk8)
            z8s.append(z8)

        ki, tie, contl, idxl = [], [], [], []
        for l in range(_K):
            zi = z8s[l][0, 0].astype(jnp.int32)
            tie.append((zi & 1) == 1)
            k = zi >> 1
            ki.append(k)
            contl.append(k < _BIGI)
            idxl.append(jnp.minimum(k >> 1, _N - 1))

        bx1l, by1l, bx2l, by2l, boffl, a1l = [], [], [], [], [], []
        for l in range(_K):
            bx1l.append(rows_ref[0, idxl[l]])
            by1l.append(rows_ref[1, idxl[l]])
            bx2l.append(rows_ref[2, idxl[l]])
            by2l.append(rows_ref[3, idxl[l]])
            boffl.append(rows_ref[4, idxl[l]] * max_coord)
            a1l.append((bx2l[l] - bx1l[l]) * (by2l[l] - by1l[l]))

        def inter_off(j, l):
            w = (jnp.minimum(bx2l[j] + boffl[j], bx2l[l] + boffl[l])
                 - jnp.maximum(bx1l[j] + boffl[j], bx1l[l] + boffl[l]))
            h = (jnp.minimum(by2l[j] + boffl[j], by2l[l] + boffl[l])
                 - jnp.maximum(by1l[j] + boffl[j], by1l[l] + boffl[l]))
            return jnp.maximum(w, 0.0) * jnp.maximum(h, 0.0)

        def hit_s(j, l):
            jw = (jnp.minimum(bx2l[j], bx2l[l])
                  - jnp.maximum(bx1l[j], bx1l[l]))
            jh = (jnp.minimum(by2l[j], by2l[l])
                  - jnp.maximum(by1l[j], by1l[l]))
            jint = jnp.maximum(jw, 0.0) * jnp.maximum(jh, 0.0)
            return ((1.0 + _IOU_THRESH) * jint
                    > _IOU_THRESH * (a1l[l] + (a1l[j] + 1e-9)))

        pairok = [None] * _K
        for l in range(1, _K):
            p = inter_off(0, l) == 0.0
            for j in range(1, l):
                p = jnp.logical_and(p, inter_off(j, l) == 0.0)
            pairok[l] = p

        proc = [contl[0]]
        for l in range(1, _K):
            proc.append(jnp.logical_and(
                jnp.logical_and(proc[l - 1], jnp.logical_not(tie[l - 1])),
                jnp.logical_and(pairok[l], contl[l])))

        takes, tcaps = [], []
        rank = kept
        for l in range(_K):
            stale = None
            for j in range(l):
                hj = jnp.logical_and(takes[j], hit_s(j, l))
                stale = hj if stale is None else jnp.logical_or(stale, hj)
            t = jnp.logical_and(proc[l], (ki[l] & 1) == 0)
            if stale is not None:
                t = jnp.logical_and(t, jnp.logical_not(stale))
            takes.append(t)
            tcaps.append(jnp.logical_and(t, rank < _DETS))
            rank = rank + t.astype(jnp.int32)

        def soft_decay(l):
            ix1 = jnp.maximum(bx1l[l] + boffl[l], xo1_ref[...])
            iy1 = jnp.maximum(by1l[l] + boffl[l], yo1_ref[...])
            ix2 = jnp.minimum(bx2l[l] + boffl[l], xo2_ref[...])
            iy2 = jnp.minimum(by2l[l] + boffl[l], yo2_ref[...])
            inter = jnp.maximum(ix2 - ix1, 0.0) * jnp.maximum(iy2 - iy1, 0.0)
            iou = inter / (a1l[l] + area - inter + 1e-9)
            return jnp.exp(-(iou * iou) / _NMS_SIGMA)

        def greedy_hit(l):
            gx1 = jnp.maximum(bx1l[l], x1)
            gy1 = jnp.maximum(by1l[l], y1)
            gx2 = jnp.minimum(bx2l[l], x2)
            gy2 = jnp.minimum(by2l[l], y2)
            gint = (jnp.maximum(gx2 - gx1, 0.0)
                    * jnp.maximum(gy2 - gy1, 0.0))
            return ((1.0 + _IOU_THRESH) * gint
                    > _IOU_THRESH * (area + (a1l[l] + 1e-9)))

        dd = soft_decay(0)
        zmask = kkey == _bcast(mk8s[0])
        sup = jnp.logical_and(greedy_hit(0), takes[0])
        out2 = jnp.where(jnp.logical_and(iotap == i, tcaps[0]),
                         _bcast(mv8s[0]), out)
        for l in range(1, _K):
            dd = dd * jnp.where(proc[l], soft_decay(l), 1.0)
            zmask = jnp.logical_or(
                zmask, jnp.logical_and(kkey == _bcast(mk8s[l]), proc[l]))
            sup = jnp.logical_or(
                sup, jnp.logical_and(greedy_hit(l), takes[l]))
            out2 = jnp.where(jnp.logical_and(iotap == i + l, tcaps[l]),
                             _bcast(mv8s[l]), out2)
        s2 = jnp.where(zmask, 0.0, s * dd)
        kkey2 = jnp.where(sup, kodd_ref[...], kkey)

        i2 = i + _K
        for l in range(_K - 1, 0, -1):
            spec_next = jnp.logical_and(jnp.logical_not(tie[l - 1]),
                                        pairok[l])
            i2 = jnp.where(spec_next,
                           jnp.where(contl[l], i2, _N),
                           i + l)
        i2 = jnp.where(contl[0], i2, _N)

        kept2 = kept
        for l in range(_K):
            kept2 = kept2 + tcaps[l].astype(jnp.int32)
        return (i2, kept2, s2, kkey2, out2)

    final = lax.while_loop(
        cond, body, (jnp.int32(0), jnp.int32(0), s0_ref[...], k0,
                     jnp.zeros((_R, 128), jnp.float32)))
    out_ref[...] = final[4]


def kernel(boxes, scores, labels):
    pad = _NP - _N
    labf = labels.astype(jnp.float32)
    coords = jnp.pad(boxes, ((0, pad), (0, 0))).T.reshape(4, _R, 128)
    rows = jnp.pad(
        jnp.concatenate([boxes.T, labf[None, :]], axis=0), ((0, 0), (0, pad)))
    sp = jnp.pad(scores, (0, pad), constant_values=-1.0).reshape(_R, 128)
    lp = jnp.pad(labf, (0, pad)).reshape(_R, 128)

    out = pl.pallas_call(
        _nms_kernel,
        out_shape=jax.ShapeDtypeStruct((_R, 128), jnp.float32),
        in_specs=[
            pl.BlockSpec(memory_space=pltpu.VMEM),
            pl.BlockSpec(memory_space=pltpu.SMEM),
            pl.BlockSpec(memory_space=pltpu.VMEM),
            pl.BlockSpec(memory_space=pltpu.VMEM),
        ],
        scratch_shapes=[pltpu.VMEM((_R, 128), jnp.float32)] * 6
                       + [pltpu.VMEM((_R, 128), jnp.int32)],
    )(coords, rows, sp, lp)
    return out.reshape(_NP)[:_N]
